# Initial kernel scaffold; baseline (speedup 1.0000x reference)
import jax
import jax.numpy as jnp
from jax.experimental import pallas as pl


def kernel(x, edge_index, batch, params):
    raise NotImplementedError("write your pallas kernel here")


# speedup vs baseline: 1.9418x; 1.9418x over previous
import functools

import jax
import jax.numpy as jnp
from jax import lax
from jax.experimental import pallas as pl
from jax.experimental.pallas import tpu as pltpu
from jax.experimental.pallas import tpu_sc as plsc

N = 10000
NP = 10240
E = 320000
EP = 327680
HID = 1024
NCB = 8
CBW = 128
NG = 256
EPS = 1e-5
RB = 512
GRID_R = NP // RB
NSC = 2
NTILE = 16
ROWS_PER_TILE = NP // NTILE
EDGES_PER_TILE = EP // NTILE
CHUNK = 128
NCHUNK = EDGES_PER_TILE // CHUNK
DUMMY = N


def _sc_deg_body(dst_hbm, ones_hbm, zeros_hbm, out_hbm, idx_d, obuf, acc, sem):
    cc = lax.axis_index("c")
    ss = lax.axis_index("s")
    row0 = ss * ROWS_PER_TILE

    pltpu.sync_copy(ones_hbm, obuf)
    pltpu.sync_copy(zeros_hbm, acc.at[pl.ds(row0, ROWS_PER_TILE)])
    plsc.subcore_barrier()

    def edge_step(k, _):
        base = cc * (EP // 2) + ss * (EP // 32) + k * CHUNK
        pltpu.sync_copy(dst_hbm.at[pl.ds(base, CHUNK)], idx_d)
        pltpu.sync_copy(obuf, acc.at[idx_d], add=True)
        return 0

    lax.fori_loop(0, (EP // 32) // CHUNK, edge_step, 0)
    plsc.subcore_barrier()
    pltpu.sync_copy(acc.at[pl.ds(row0, ROWS_PER_TILE)],
                    out_hbm.at[pl.ds(cc * NP + row0, ROWS_PER_TILE)])


def _sc_deg(dst_p, ones_blk, zeros_blk):
    mesh = plsc.VectorSubcoreMesh(core_axis_name="c", subcore_axis_name="s")
    fn = pl.kernel(
        _sc_deg_body,
        out_type=jax.ShapeDtypeStruct((NSC * NP, CBW), jnp.float32),
        mesh=mesh,
        scratch_types=[
            pltpu.VMEM((CHUNK,), jnp.int32),
            pltpu.VMEM((CHUNK, CBW), jnp.float32),
            pltpu.VMEM_SHARED((NP, CBW), jnp.float32),
            pltpu.SemaphoreType.DMA,
        ],
    )
    return fn(dst_p, ones_blk, zeros_blk)


def _sc_agg_body(m_hbm, src_hbm, dst_hbm, zeros_hbm, out_hbm,
                 idx_s, idx_d, gbuf, acc, sem):
    cc = lax.axis_index("c")
    ss = lax.axis_index("s")
    row0 = ss * ROWS_PER_TILE

    for cb4 in range(NCB // NSC):
        cbg = cc * (NCB // NSC) + cb4
        cboff = cbg * NP

        pltpu.sync_copy(zeros_hbm, acc.at[pl.ds(row0, ROWS_PER_TILE)])
        plsc.subcore_barrier()

        def edge_step(k, _):
            base = ss * EDGES_PER_TILE + k * CHUNK
            pltpu.sync_copy(src_hbm.at[pl.ds(base, CHUNK)], idx_s)
            pltpu.sync_copy(dst_hbm.at[pl.ds(base, CHUNK)], idx_d)
            for j in range(CHUNK // 16):
                idx_s[pl.ds(j * 16, 16)] = idx_s[pl.ds(j * 16, 16)] + cboff
            pltpu.async_copy(m_hbm.at[idx_s], gbuf, sem).wait()
            pltpu.sync_copy(gbuf, acc.at[idx_d], add=True)
            return 0

        lax.fori_loop(0, NCHUNK, edge_step, 0)
        plsc.subcore_barrier()

        pltpu.sync_copy(acc.at[pl.ds(row0, ROWS_PER_TILE)],
                        out_hbm.at[pl.ds(cboff + row0, ROWS_PER_TILE)])


def _sc_agg(m_flat, src_p, dst_p, zeros_blk):
    mesh = plsc.VectorSubcoreMesh(core_axis_name="c", subcore_axis_name="s")
    fn = pl.kernel(
        _sc_agg_body,
        out_type=jax.ShapeDtypeStruct((NCB * NP, CBW), jnp.float32),
        mesh=mesh,
        scratch_types=[
            pltpu.VMEM((CHUNK,), jnp.int32),
            pltpu.VMEM((CHUNK,), jnp.int32),
            pltpu.VMEM((CHUNK, CBW), jnp.float32),
            pltpu.VMEM_SHARED((NP, CBW), jnp.float32),
            pltpu.SemaphoreType.DMA,
        ],
    )
    return fn(m_flat, src_p, dst_p, zeros_blk)


def _k_dis_body(p_ref, o_ref):
    s = p_ref[0:NP, 0:1] + p_ref[NP:2 * NP, 0:1]
    row = lax.broadcasted_iota(jnp.int32, (NP, 1), 0)
    o_ref[...] = jnp.where(row < N, lax.rsqrt(s + 1.0), 0.0)


def _k_dis(partials):
    return pl.pallas_call(
        _k_dis_body,
        out_shape=jax.ShapeDtypeStruct((NP, 1), jnp.float32),
    )(partials)


def _k_emb_body(x_ref, w_ref, b_ref, o_ref):
    z = jnp.dot(x_ref[...], w_ref[...], preferred_element_type=jnp.float32)
    z = z + b_ref[...]
    o_ref[...] = jnp.where(z > 0, z, jnp.exp(jnp.minimum(z, 0.0)) - 1.0)


def _k_emb(x_p, emb_W, emb_b):
    return pl.pallas_call(
        _k_emb_body,
        grid=(GRID_R,),
        in_specs=[
            pl.BlockSpec((RB, 128), lambda i: (i, 0)),
            pl.BlockSpec((128, HID), lambda i: (0, 0)),
            pl.BlockSpec((1, HID), lambda i: (0, 0)),
        ],
        out_specs=pl.BlockSpec((RB, HID), lambda i: (i, 0)),
        out_shape=jax.ShapeDtypeStruct((NP, HID), jnp.float32),
    )(x_p, emb_W, emb_b)


def _k_mm_body(h_ref, a_ref, c_ref, w_ref, d_ref, o_ref, *, act):
    h = h_ref[...]
    if act:
        h = jnp.maximum(h * a_ref[...] + c_ref[...], 0.0)
    m = jnp.dot(h, w_ref[...], preferred_element_type=jnp.float32)
    m = m * d_ref[...]
    for cb in range(NCB):
        o_ref[cb] = m[:, cb * CBW:(cb + 1) * CBW]


def _k_mm(h, a, c, W, dis_col, act):
    return pl.pallas_call(
        functools.partial(_k_mm_body, act=act),
        grid=(GRID_R,),
        in_specs=[
            pl.BlockSpec((RB, HID), lambda i: (i, 0)),
            pl.BlockSpec((1, HID), lambda i: (0, 0)),
            pl.BlockSpec((1, HID), lambda i: (0, 0)),
            pl.BlockSpec((HID, HID), lambda i: (0, 0)),
            pl.BlockSpec((RB, 1), lambda i: (i, 0)),
        ],
        out_specs=pl.BlockSpec((NCB, RB, CBW), lambda i: (0, i, 0)),
        out_shape=jax.ShapeDtypeStruct((NCB, NP, CBW), jnp.float32),
    )(h, a, c, W, dis_col)


def _k_comb_body(n_ref, m_ref, d_ref, b_ref, h_ref, s_ref):
    i = pl.program_id(0)
    parts = [n_ref[cb] + m_ref[cb] for cb in range(NCB)]
    s = jnp.concatenate(parts, axis=1)
    hp = s * d_ref[...] + b_ref[...]
    h_ref[...] = hp
    rowid = lax.broadcasted_iota(jnp.int32, (RB, 1), 0) + i * RB
    hpm = jnp.where(rowid < N, hp, 0.0)
    su = jnp.sum(hpm, axis=0, keepdims=True)
    sq = jnp.sum(hpm * hpm, axis=0, keepdims=True)

    @pl.when(i == 0)
    def _():
        s_ref[...] = jnp.zeros((8, HID), jnp.float32)

    s_ref[0:1, :] = s_ref[0:1, :] + su
    s_ref[1:2, :] = s_ref[1:2, :] + sq


def _k_comb(neigh, mblk, dis_col, bvec):
    return pl.pallas_call(
        _k_comb_body,
        grid=(GRID_R,),
        in_specs=[
            pl.BlockSpec((NCB, RB, CBW), lambda i: (0, i, 0)),
            pl.BlockSpec((NCB, RB, CBW), lambda i: (0, i, 0)),
            pl.BlockSpec((RB, 1), lambda i: (i, 0)),
            pl.BlockSpec((1, HID), lambda i: (0, 0)),
        ],
        out_specs=[
            pl.BlockSpec((RB, HID), lambda i: (i, 0)),
            pl.BlockSpec((8, HID), lambda i: (0, 0)),
        ],
        out_shape=[
            jax.ShapeDtypeStruct((NP, HID), jnp.float32),
            jax.ShapeDtypeStruct((8, HID), jnp.float32),
        ],
    )(neigh, mblk, dis_col, bvec)


def _k_ac_body(s_ref, g_ref, b_ref, a_ref, c_ref):
    mean = s_ref[0:1, :] * (1.0 / N)
    ex2 = s_ref[1:2, :] * (1.0 / N)
    var = ex2 - mean * mean
    a = g_ref[...] * lax.rsqrt(var + EPS)
    a_ref[...] = a
    c_ref[...] = b_ref[...] - mean * a


def _k_ac(stats, g, bb):
    return pl.pallas_call(
        _k_ac_body,
        out_shape=[
            jax.ShapeDtypeStruct((1, HID), jnp.float32),
            jax.ShapeDtypeStruct((1, HID), jnp.float32),
        ],
    )(stats, g, bb)


def _k_pool_body(h_ref, a_ref, c_ref, bt_ref, o_ref):
    i = pl.program_id(0)
    t = jnp.maximum(h_ref[...] * a_ref[...] + c_ref[...], 0.0)
    gid = lax.broadcasted_iota(jnp.int32, (NG, RB), 0)
    sel = (bt_ref[...] == gid).astype(jnp.float32)
    contrib = jnp.dot(sel, t, preferred_element_type=jnp.float32)

    @pl.when(i == 0)
    def _():
        o_ref[...] = jnp.zeros((NG, HID), jnp.float32)

    o_ref[...] = o_ref[...] + contrib


def _k_pool(hpre, a, c, batch2d):
    return pl.pallas_call(
        _k_pool_body,
        grid=(GRID_R,),
        in_specs=[
            pl.BlockSpec((RB, HID), lambda i: (i, 0)),
            pl.BlockSpec((1, HID), lambda i: (0, 0)),
            pl.BlockSpec((1, HID), lambda i: (0, 0)),
            pl.BlockSpec((1, RB), lambda i: (0, i)),
        ],
        out_specs=pl.BlockSpec((NG, HID), lambda i: (0, 0)),
        out_shape=jax.ShapeDtypeStruct((NG, HID), jnp.float32),
    )(hpre, a, c, batch2d)


def _k_fc_body(hg_ref, w0, w1, w2, b0, b1, b2, g0, g1, g2,
               q0, q1, q2, wo, bo, o_ref):
    h = hg_ref[...]
    for w, b, g, q in ((w0, b0, g0, q0), (w1, b1, g1, q1), (w2, b2, g2, q2)):
        z = jnp.dot(h, w[...], preferred_element_type=jnp.float32) + b[...]
        mean = jnp.sum(z, axis=0, keepdims=True) * (1.0 / NG)
        var = jnp.sum(z * z, axis=0, keepdims=True) * (1.0 / NG) - mean * mean
        z = g[...] * (z - mean) * lax.rsqrt(var + EPS) + q[...]
        h = jnp.maximum(z, 0.0)
    zo = jnp.dot(h, wo[...], preferred_element_type=jnp.float32) + bo[...]
    z0 = zo[:, 0:1]
    z1 = zo[:, 1:2]
    mx = jnp.maximum(z0, z1)
    lse = mx + jnp.log(jnp.exp(z0 - mx) + jnp.exp(z1 - mx))
    o_ref[...] = zo - lse


def _k_fc(hg, fw, fb, fg, fq, wo_p, bo_p):
    return pl.pallas_call(
        _k_fc_body,
        out_shape=jax.ShapeDtypeStruct((NG, 128), jnp.float32),
    )(hg, fw[0], fw[1], fw[2], fb[0], fb[1], fb[2],
      fg[0], fg[1], fg[2], fq[0], fq[1], fq[2], wo_p, bo_p)


def kernel(x, edge_index, batch, params):
    f32 = jnp.float32
    src = edge_index[0].astype(jnp.int32)
    dst = edge_index[1].astype(jnp.int32)
    src_p = jnp.concatenate([src, jnp.zeros((EP - E,), jnp.int32)])
    dst_p = jnp.concatenate([dst, jnp.full((EP - E,), DUMMY, jnp.int32)])
    x_p = jnp.pad(x, ((0, NP - N), (0, 0)))
    batch2d = jnp.pad(batch.astype(jnp.int32), (0, NP - N),
                      constant_values=NG).reshape(1, NP)
    zeros_blk = jnp.zeros((ROWS_PER_TILE, CBW), f32)
    ones_blk = jnp.ones((CHUNK, CBW), f32)

    partials = _sc_deg(dst_p, ones_blk, zeros_blk)
    dis_col = _k_dis(partials)

    h = _k_emb(x_p, params['emb_W'], params['emb_b'].reshape(1, HID))

    ones_a = jnp.ones((1, HID), f32)
    zeros_c = jnp.zeros((1, HID), f32)
    a_vec, c_vec = ones_a, zeros_c
    for li in range(5):
        W = params['conv_W'][li]
        bvec = params['conv_b'][li].reshape(1, HID)
        g = params['bn_g'][li].reshape(1, HID)
        q = params['bn_b'][li].reshape(1, HID)
        mblk = _k_mm(h, a_vec, c_vec, W, dis_col, act=(li > 0))
        m_flat = mblk.reshape(NCB * NP, CBW)
        neigh_flat = _sc_agg(m_flat, src_p, dst_p, zeros_blk)
        neigh = neigh_flat.reshape(NCB, NP, CBW)
        h, stats = _k_comb(neigh, mblk, dis_col, bvec)
        a_vec, c_vec = _k_ac(stats, g, q)

    hg = _k_pool(h, a_vec, c_vec, batch2d)

    fw = params['fc_W']
    fb = [b.reshape(1, HID) for b in params['fc_b']]
    fg = [g.reshape(1, HID) for g in params['fcn_g']]
    fq = [q.reshape(1, HID) for q in params['fcn_b']]
    wo_p = jnp.pad(params['out_W'], ((0, 0), (0, 128 - 2)))
    bo_p = jnp.pad(params['out_b'], (0, 128 - 2)).reshape(1, 128)
    out = _k_fc(hg, fw, fb, fg, fq, wo_p, bo_p)
    return out[:, :2]


# speedup vs baseline: 2.6649x; 1.3724x over previous
import functools

import jax
import jax.numpy as jnp
from jax import lax
from jax.experimental import pallas as pl
from jax.experimental.pallas import tpu as pltpu
from jax.experimental.pallas import tpu_sc as plsc

N = 10000
NP = 10240
E = 320000
EP = 327680
HID = 1024
NCB = 8
CBW = 128
NG = 256
EPS = 1e-5
RB = 512
GRID_R = NP // RB
NSC = 2
NTILE = 16
ROWS_PER_TILE = NP // NTILE
EDGES_PER_TILE = EP // NTILE
CHUNK = 64
NCHUNK = EDGES_PER_TILE // CHUNK
DUMMY = N
ACC_ROWS = 10112
ACC_RPT = ACC_ROWS // NTILE


DCHUNK = 128


def _sc_deg_body(dst_hbm, ones_hbm, zeros_hbm, out_hbm, idx_d, obuf, acc, sem):
    cc = lax.axis_index("c")
    ss = lax.axis_index("s")
    row0 = ss * ROWS_PER_TILE

    pltpu.sync_copy(ones_hbm, obuf)
    pltpu.sync_copy(zeros_hbm, acc.at[pl.ds(row0, ROWS_PER_TILE)])
    plsc.subcore_barrier()

    def edge_step(k, _):
        base = cc * (EP // 2) + ss * (EP // 32) + k * DCHUNK
        pltpu.sync_copy(dst_hbm.at[pl.ds(base, DCHUNK)], idx_d)
        pltpu.sync_copy(obuf, acc.at[idx_d], add=True)
        return 0

    lax.fori_loop(0, (EP // 32) // DCHUNK, edge_step, 0)
    plsc.subcore_barrier()
    pltpu.sync_copy(acc.at[pl.ds(row0, ROWS_PER_TILE)],
                    out_hbm.at[pl.ds(cc * NP + row0, ROWS_PER_TILE)])


def _sc_deg(dst_p, ones_blk, zeros_blk):
    mesh = plsc.VectorSubcoreMesh(core_axis_name="c", subcore_axis_name="s")
    fn = pl.kernel(
        _sc_deg_body,
        out_type=jax.ShapeDtypeStruct((NSC * NP, CBW), jnp.float32),
        mesh=mesh,
        scratch_types=[
            pltpu.VMEM((DCHUNK,), jnp.int32),
            pltpu.VMEM((DCHUNK, CBW), jnp.float32),
            pltpu.VMEM_SHARED((NP, CBW), jnp.float32),
            pltpu.SemaphoreType.DMA,
        ],
    )
    return fn(dst_p, ones_blk, zeros_blk)


GPC = 8
NGRP = NCHUNK // GPC
SGRP_S = (EP // CHUNK) // GPC
TGRP = NGRP // 4


def _sc_agg_body(m_hbm, src8_hbm, dst2_hbm, zeros_hbm, out_hbm,
                 sidx, didx, g0, g1, g2, g3, acc, *sems):
    cc = lax.axis_index("c")
    ss = lax.axis_index("s")
    row0 = ss * ACC_RPT
    g = (g0, g1, g2, g3)
    isem = sems[0:4]
    gs = sems[4:8]
    ts = sems[8:12]

    def fire_grp(cbg, gi, gb):
        srow = cbg * SGRP_S + ss * NGRP + gi
        drow = ss * NGRP + gi
        pltpu.async_copy(src8_hbm.at[srow], sidx.at[gb], isem[gb])
        pltpu.async_copy(dst2_hbm.at[drow], didx.at[gb], isem[gb])

    def wait_grp(cbg, gi, gb):
        srow = cbg * SGRP_S + ss * NGRP + gi
        drow = ss * NGRP + gi
        pltpu.make_async_copy(src8_hbm.at[srow], sidx.at[gb],
                              isem[gb]).wait()
        pltpu.make_async_copy(dst2_hbm.at[drow], didx.at[gb],
                              isem[gb]).wait()

    def wait_gather(b):
        pltpu.make_async_copy(m_hbm.at[sidx.at[0].at[0]], g[b], gs[b]).wait()

    def fire_scatter(b, gb, j):
        pltpu.async_copy(g[b], acc.at[didx.at[gb].at[j]], ts[b], add=True)

    def wait_scatter(b):
        pltpu.make_async_copy(g[b], acc.at[didx.at[0].at[0]], ts[b]).wait()

    for cb4 in range(NCB // NSC):
        cbg = cc * (NCB // NSC) + cb4

        pltpu.sync_copy(zeros_hbm, acc.at[pl.ds(row0, ACC_RPT)])
        fire_grp(cbg, 0, 0)
        fire_grp(cbg, 1, 1)
        plsc.subcore_barrier()

        def super_grp(sg, _):
            for gg in range(4):
                gi = sg * 4 + gg
                wait_grp(cbg, gi, gg)

                @pl.when(gi + 2 < NGRP)
                def _():
                    fire_grp(cbg, gi + 2, (gg + 2) % 4)
                for j in range(GPC):
                    k = gi * GPC + j
                    b = j % 4

                    @pl.when(k - 4 >= 0)
                    def _():
                        wait_scatter(b)
                    pltpu.async_copy(m_hbm.at[sidx.at[gg].at[j]], g[b],
                                     gs[b])

                    @pl.when(k - 2 >= 0)
                    def _():
                        wait_gather((j + 2) % 4)
                        if j >= 2:
                            fire_scatter((j + 2) % 4, gg, j - 2)
                        else:
                            fire_scatter((j + 2) % 4, (gg + 3) % 4,
                                         j + GPC - 2)
            return 0

        lax.fori_loop(0, TGRP, super_grp, 0)
        wait_gather(2)
        fire_scatter(2, 3, GPC - 2)
        wait_gather(3)
        fire_scatter(3, 3, GPC - 1)
        for b in range(4):
            wait_scatter(b)
        plsc.subcore_barrier()

        pltpu.sync_copy(acc.at[pl.ds(row0, ACC_RPT)],
                        out_hbm.at[pl.ds(cbg * NP + row0, ACC_RPT)])


def _sc_agg(m_flat, src8_3d, dst3d, zeros_acc):
    mesh = plsc.VectorSubcoreMesh(core_axis_name="c", subcore_axis_name="s")
    fn = pl.kernel(
        _sc_agg_body,
        out_type=jax.ShapeDtypeStruct((NCB * NP, CBW), jnp.float32),
        mesh=mesh,
        scratch_types=[
            pltpu.VMEM((4, GPC, CHUNK), jnp.int32),
            pltpu.VMEM((4, GPC, CHUNK), jnp.int32),
            pltpu.VMEM((CHUNK, CBW), jnp.float32),
            pltpu.VMEM((CHUNK, CBW), jnp.float32),
            pltpu.VMEM((CHUNK, CBW), jnp.float32),
            pltpu.VMEM((CHUNK, CBW), jnp.float32),
            pltpu.VMEM_SHARED((ACC_ROWS, CBW), jnp.float32),
        ] + [pltpu.SemaphoreType.DMA] * 12,
    )
    return fn(m_flat, src8_3d, dst3d, zeros_acc)


def _k_dis_body(p_ref, o_ref):
    s = p_ref[0:NP, 0:1] + p_ref[NP:2 * NP, 0:1]
    row = lax.broadcasted_iota(jnp.int32, (NP, 1), 0)
    o_ref[...] = jnp.where(row < N, lax.rsqrt(s + 1.0), 0.0)


def _k_dis(partials):
    return pl.pallas_call(
        _k_dis_body,
        out_shape=jax.ShapeDtypeStruct((NP, 1), jnp.float32),
    )(partials)


def _k_emb_body(x_ref, w_ref, b_ref, o_ref):
    z = jnp.dot(x_ref[...], w_ref[...], preferred_element_type=jnp.float32, precision=lax.Precision.HIGHEST)
    z = z + b_ref[...]
    o_ref[...] = jnp.where(z > 0, z, jnp.exp(jnp.minimum(z, 0.0)) - 1.0)


def _k_emb(x_p, emb_W, emb_b):
    return pl.pallas_call(
        _k_emb_body,
        grid=(GRID_R,),
        in_specs=[
            pl.BlockSpec((RB, 128), lambda i: (i, 0)),
            pl.BlockSpec((128, HID), lambda i: (0, 0)),
            pl.BlockSpec((1, HID), lambda i: (0, 0)),
        ],
        out_specs=pl.BlockSpec((RB, HID), lambda i: (i, 0)),
        out_shape=jax.ShapeDtypeStruct((NP, HID), jnp.float32),
    )(x_p, emb_W, emb_b)


def _k_mm_body(h_ref, a_ref, c_ref, w_ref, d_ref, o_ref, *, act):
    h = h_ref[...]
    if act:
        h = jnp.maximum(h * a_ref[...] + c_ref[...], 0.0)
    m = jnp.dot(h, w_ref[...], preferred_element_type=jnp.float32, precision=lax.Precision.HIGHEST)
    m = m * d_ref[...]
    for cb in range(NCB):
        o_ref[cb] = m[:, cb * CBW:(cb + 1) * CBW]


def _k_mm(h, a, c, W, dis_col, act):
    return pl.pallas_call(
        functools.partial(_k_mm_body, act=act),
        grid=(GRID_R,),
        in_specs=[
            pl.BlockSpec((RB, HID), lambda i: (i, 0)),
            pl.BlockSpec((1, HID), lambda i: (0, 0)),
            pl.BlockSpec((1, HID), lambda i: (0, 0)),
            pl.BlockSpec((HID, HID), lambda i: (0, 0)),
            pl.BlockSpec((RB, 1), lambda i: (i, 0)),
        ],
        out_specs=pl.BlockSpec((NCB, RB, CBW), lambda i: (0, i, 0)),
        out_shape=jax.ShapeDtypeStruct((NCB, NP, CBW), jnp.float32),
    )(h, a, c, W, dis_col)


def _k_comb_body(n_ref, m_ref, d_ref, b_ref, h_ref, s_ref):
    i = pl.program_id(0)
    parts = [n_ref[cb] + m_ref[cb] for cb in range(NCB)]
    s = jnp.concatenate(parts, axis=1)
    hp = s * d_ref[...] + b_ref[...]
    rowid = lax.broadcasted_iota(jnp.int32, (RB, 1), 0) + i * RB
    hpm = jnp.where(rowid < N, hp, 0.0)
    h_ref[...] = hpm
    su = jnp.sum(hpm, axis=0, keepdims=True)
    sq = jnp.sum(hpm * hpm, axis=0, keepdims=True)

    @pl.when(i == 0)
    def _():
        s_ref[...] = jnp.zeros((8, HID), jnp.float32)

    s_ref[0:1, :] = s_ref[0:1, :] + su
    s_ref[1:2, :] = s_ref[1:2, :] + sq


def _k_comb(neigh, mblk, dis_col, bvec):
    return pl.pallas_call(
        _k_comb_body,
        grid=(GRID_R,),
        in_specs=[
            pl.BlockSpec((NCB, RB, CBW), lambda i: (0, i, 0)),
            pl.BlockSpec((NCB, RB, CBW), lambda i: (0, i, 0)),
            pl.BlockSpec((RB, 1), lambda i: (i, 0)),
            pl.BlockSpec((1, HID), lambda i: (0, 0)),
        ],
        out_specs=[
            pl.BlockSpec((RB, HID), lambda i: (i, 0)),
            pl.BlockSpec((8, HID), lambda i: (0, 0)),
        ],
        out_shape=[
            jax.ShapeDtypeStruct((NP, HID), jnp.float32),
            jax.ShapeDtypeStruct((8, HID), jnp.float32),
        ],
    )(neigh, mblk, dis_col, bvec)


def _k_ac_body(s_ref, g_ref, b_ref, a_ref, c_ref):
    mean = s_ref[0:1, :] * (1.0 / N)
    ex2 = s_ref[1:2, :] * (1.0 / N)
    var = ex2 - mean * mean
    a = g_ref[...] * lax.rsqrt(var + EPS)
    a_ref[...] = a
    c_ref[...] = b_ref[...] - mean * a


def _k_ac(stats, g, bb):
    return pl.pallas_call(
        _k_ac_body,
        out_shape=[
            jax.ShapeDtypeStruct((1, HID), jnp.float32),
            jax.ShapeDtypeStruct((1, HID), jnp.float32),
        ],
    )(stats, g, bb)


def _k_pool_body(h_ref, a_ref, c_ref, bt_ref, o_ref):
    i = pl.program_id(0)
    t = jnp.maximum(h_ref[...] * a_ref[...] + c_ref[...], 0.0)
    gid = lax.broadcasted_iota(jnp.int32, (NG, RB), 0)
    sel = (bt_ref[...] == gid).astype(jnp.float32)
    contrib = jnp.dot(sel, t, preferred_element_type=jnp.float32, precision=lax.Precision.HIGHEST)

    @pl.when(i == 0)
    def _():
        o_ref[...] = jnp.zeros((NG, HID), jnp.float32)

    o_ref[...] = o_ref[...] + contrib


def _k_pool(hpre, a, c, batch2d):
    return pl.pallas_call(
        _k_pool_body,
        grid=(GRID_R,),
        in_specs=[
            pl.BlockSpec((RB, HID), lambda i: (i, 0)),
            pl.BlockSpec((1, HID), lambda i: (0, 0)),
            pl.BlockSpec((1, HID), lambda i: (0, 0)),
            pl.BlockSpec((1, RB), lambda i: (0, i)),
        ],
        out_specs=pl.BlockSpec((NG, HID), lambda i: (0, 0)),
        out_shape=jax.ShapeDtypeStruct((NG, HID), jnp.float32),
    )(hpre, a, c, batch2d)


def _k_fc_body(hg_ref, w0, w1, w2, b0, b1, b2, g0, g1, g2,
               q0, q1, q2, wo, bo, o_ref):
    h = hg_ref[...]
    for w, b, g, q in ((w0, b0, g0, q0), (w1, b1, g1, q1), (w2, b2, g2, q2)):
        z = jnp.dot(h, w[...], preferred_element_type=jnp.float32, precision=lax.Precision.HIGHEST) + b[...]
        mean = jnp.sum(z, axis=0, keepdims=True) * (1.0 / NG)
        var = jnp.sum(z * z, axis=0, keepdims=True) * (1.0 / NG) - mean * mean
        z = g[...] * (z - mean) * lax.rsqrt(var + EPS) + q[...]
        h = jnp.maximum(z, 0.0)
    zo = jnp.dot(h, wo[...], preferred_element_type=jnp.float32, precision=lax.Precision.HIGHEST) + bo[...]
    z0 = zo[:, 0:1]
    z1 = zo[:, 1:2]
    mx = jnp.maximum(z0, z1)
    lse = mx + jnp.log(jnp.exp(z0 - mx) + jnp.exp(z1 - mx))
    o_ref[...] = zo - lse


def _k_fc(hg, fw, fb, fg, fq, wo_p, bo_p):
    return pl.pallas_call(
        _k_fc_body,
        out_shape=jax.ShapeDtypeStruct((NG, 128), jnp.float32),
    )(hg, fw[0], fw[1], fw[2], fb[0], fb[1], fb[2],
      fg[0], fg[1], fg[2], fq[0], fq[1], fq[2], wo_p, bo_p)


def kernel(x, edge_index, batch, params):
    f32 = jnp.float32
    src = edge_index[0].astype(jnp.int32)
    dst = edge_index[1].astype(jnp.int32)
    src_p = jnp.concatenate([src, jnp.zeros((EP - E,), jnp.int32)])
    dst_p = jnp.concatenate([dst, jnp.full((EP - E,), DUMMY, jnp.int32)])
    src8_3d = (jnp.arange(NCB, dtype=jnp.int32)[:, None] * NP
               + src_p[None, :]).reshape(NCB * EP // (CHUNK * GPC), GPC, CHUNK)
    dst3d = dst_p.reshape(EP // (CHUNK * GPC), GPC, CHUNK)
    x_p = jnp.pad(x, ((0, NP - N), (0, 0)))
    batch2d = jnp.pad(batch.astype(jnp.int32), (0, NP - N),
                      constant_values=NG).reshape(1, NP)
    zeros_blk = jnp.zeros((ROWS_PER_TILE, CBW), f32)
    zeros_acc = jnp.zeros((ACC_RPT, CBW), f32)
    ones_blk = jnp.ones((DCHUNK, CBW), f32)

    partials = _sc_deg(dst_p, ones_blk, zeros_blk)
    dis_col = _k_dis(partials)

    h = _k_emb(x_p, params['emb_W'], params['emb_b'].reshape(1, HID))

    ones_a = jnp.ones((1, HID), f32)
    zeros_c = jnp.zeros((1, HID), f32)
    a_vec, c_vec = ones_a, zeros_c
    for li in range(5):
        W = params['conv_W'][li]
        bvec = params['conv_b'][li].reshape(1, HID)
        g = params['bn_g'][li].reshape(1, HID)
        q = params['bn_b'][li].reshape(1, HID)
        mblk = _k_mm(h, a_vec, c_vec, W, dis_col, act=(li > 0))
        m_flat = mblk.reshape(NCB * NP, CBW)
        neigh_flat = _sc_agg(m_flat, src8_3d, dst3d, zeros_acc)
        neigh = neigh_flat.reshape(NCB, NP, CBW)
        h, stats = _k_comb(neigh, mblk, dis_col, bvec)
        a_vec, c_vec = _k_ac(stats, g, q)

    hg = _k_pool(h, a_vec, c_vec, batch2d)

    fw = params['fc_W']
    fb = [b.reshape(1, HID) for b in params['fc_b']]
    fg = [g.reshape(1, HID) for g in params['fcn_g']]
    fq = [q.reshape(1, HID) for q in params['fcn_b']]
    wo_p = jnp.pad(params['out_W'], ((0, 0), (0, 128 - 2)))
    bo_p = jnp.pad(params['out_b'], (0, 128 - 2)).reshape(1, 128)
    out = _k_fc(hg, fw, fb, fg, fq, wo_p, bo_p)
    return out[:, :2]
